# initial kernel scaffold (unmeasured)
import jax
import jax.numpy as jnp
from jax import lax
from jax.experimental import pallas as pl
from jax.experimental.pallas import tpu as pltpu

N_DEV = 32
B, S, D = 2, 512, 2048
H, Dh, Dr = 16, 128, 32
ROWS = 4 * S
CHUNK = ROWS // N_DEV
SCALE = (Dh + Dr) ** -0.5

BF = jnp.bfloat16
F32 = jnp.float32


def _dot(a, b):
    return lax.dot(a, b, preferred_element_type=F32)


def _dot_t(a, b):
    return lax.dot_general(a, b, (((1,), (1,)), ((), ())),
                           preferred_element_type=F32)


def _comm_body(x_ref, wdkv_ref, wuk_ref, wuv_ref, kv_ref, rs_recv,
               rs_send_sems, rs_recv_sems, ag_send_sems, ag_recv_sems):
    my = lax.axis_index("i")
    left = lax.rem(my + N_DEV - 1, N_DEV)
    right = lax.rem(my + 1, N_DEV)

    wdkv = wdkv_ref[:].astype(BF)
    wuk = wuk_ref[:].astype(BF)
    wuv = wuv_ref[:].astype(BF)
    for b in range(B):
        xb = x_ref[b].astype(BF)
        cb = _dot(xb, wdkv).astype(BF)
        kb = _dot(cb, wuk).astype(BF)
        vb = _dot(cb, wuv).astype(BF)
        n_chunks = S // CHUNK
        kv_ref[pl.ds(b * n_chunks, n_chunks)] = kb.reshape(n_chunks, CHUNK, D)
        kv_ref[pl.ds((2 + b) * n_chunks, n_chunks)] = vb.reshape(
            n_chunks, CHUNK, D)

    barrier_sem = pltpu.get_barrier_semaphore()
    for nbr in (left, right):
        pl.semaphore_signal(barrier_sem, inc=1, device_id=(nbr,),
                            device_id_type=pl.DeviceIdType.MESH)
    pl.semaphore_wait(barrier_sem, 2)

    for s in range(N_DEV - 1):
        send_idx = lax.rem(my - s + N_DEV, N_DEV)
        recv_idx = lax.rem(my - s - 1 + N_DEV, N_DEV)
        rdma = pltpu.make_async_remote_copy(
            src_ref=kv_ref.at[send_idx],
            dst_ref=rs_recv.at[s],
            send_sem=rs_send_sems.at[s],
            recv_sem=rs_recv_sems.at[s],
            device_id=(right,),
            device_id_type=pl.DeviceIdType.MESH,
        )
        rdma.start()
        rdma.wait()
        kv_ref[recv_idx] = (kv_ref[recv_idx].astype(F32)
                            + rs_recv[s].astype(F32)).astype(BF)

    for s in range(N_DEV - 1):
        send_idx = lax.rem(my + 1 - s + N_DEV, N_DEV)
        rdma = pltpu.make_async_remote_copy(
            src_ref=kv_ref.at[send_idx],
            dst_ref=kv_ref.at[send_idx],
            send_sem=ag_send_sems.at[s],
            recv_sem=ag_recv_sems.at[s],
            device_id=(right,),
            device_id_type=pl.DeviceIdType.MESH,
        )
        rdma.start()
        rdma.wait()


def _allreduce_kv(x, wdkv, wuk, wuv):
    return pl.pallas_call(
        _comm_body,
        out_shape=jax.ShapeDtypeStruct((N_DEV, CHUNK, D), BF),
        in_specs=[pl.BlockSpec(memory_space=pltpu.VMEM)] * 4,
        out_specs=pl.BlockSpec(memory_space=pltpu.VMEM),
        scratch_shapes=[
            pltpu.VMEM((N_DEV - 1, CHUNK, D), BF),
            pltpu.SemaphoreType.DMA((N_DEV - 1,)),
            pltpu.SemaphoreType.DMA((N_DEV - 1,)),
            pltpu.SemaphoreType.DMA((N_DEV - 1,)),
            pltpu.SemaphoreType.DMA((N_DEV - 1,)),
        ],
        compiler_params=pltpu.CompilerParams(collective_id=0),
    )(x, wdkv, wuk, wuv)


def _attn_body(x_ref, wq_ref, wqr_ref, wkr_ref, kv_ref, o_ref,
               q_scr, qr_scr, kr_scr):
    wq = wq_ref[:].astype(BF)
    wqr = wqr_ref[:].astype(BF)
    wkr = wkr_ref[:].astype(BF)
    for b in range(B):
        xb = x_ref[b].astype(BF)
        q_scr[...] = _dot(xb, wq).astype(BF)
        qr_scr[...] = _dot(xb, wqr).astype(BF)
        kr_scr[...] = _dot(xb, wkr).astype(BF)
        kr = kr_scr[...]
        for h in range(H):
            qh = q_scr[:, h * Dh:(h + 1) * Dh]
            kh = kv_ref[b, :, h * Dh:(h + 1) * Dh]
            qrh = qr_scr[:, h * Dr:(h + 1) * Dr]
            sc = (_dot_t(qh, kh) + _dot_t(qrh, kr)) * SCALE
            m = jnp.max(sc, axis=-1, keepdims=True)
            p = jnp.exp(sc - m)
            p = p / jnp.sum(p, axis=-1, keepdims=True)
            vh = kv_ref[2 + b, :, h * Dh:(h + 1) * Dh]
            o_ref[b, :, h * Dh:(h + 1) * Dh] = _dot(p.astype(BF),
                                                    vh).astype(BF)


def _attention(x, wq, wqr, wkr, kv):
    return pl.pallas_call(
        _attn_body,
        out_shape=jax.ShapeDtypeStruct((B, S, H * Dh), BF),
        in_specs=[pl.BlockSpec(memory_space=pltpu.VMEM)] * 5,
        out_specs=pl.BlockSpec(memory_space=pltpu.VMEM),
        scratch_shapes=[
            pltpu.VMEM((S, H * Dh), BF),
            pltpu.VMEM((S, H * Dr), BF),
            pltpu.VMEM((S, Dr), BF),
        ],
    )(x, wq, wqr, wkr, kv)


def _proj_body(o_ref, wo_ref, out_ref):
    wo = wo_ref[:].astype(BF)
    for b in range(B):
        out_ref[b] = _dot(o_ref[b], wo)


def _out_proj(o, wo):
    return pl.pallas_call(
        _proj_body,
        out_shape=jax.ShapeDtypeStruct((B, S, D), F32),
        in_specs=[pl.BlockSpec(memory_space=pltpu.VMEM)] * 2,
        out_specs=pl.BlockSpec(memory_space=pltpu.VMEM),
    )(o, wo)


def kernel(x, Wdkv, Wuk, Wuv, Wq, Wqr, Wkr, Wo):
    kv = _allreduce_kv(x, Wdkv, Wuk, Wuv)
    kv = kv.reshape(4, S, D)
    o = _attention(x, Wq, Wqr, Wkr, kv)
    return _out_proj(o, Wo)


# baseline (device time: 381931 ns/iter reference)
import jax
import jax.numpy as jnp
from jax import lax
from jax.experimental import pallas as pl
from jax.experimental.pallas import tpu as pltpu

N_DEV = 32
B, S, D = 2, 512, 2048
H, Dh, Dr = 16, 128, 32
ROWS = 4 * S
CHUNK = ROWS // N_DEV
SCALE = (Dh + Dr) ** -0.5

BF = jnp.bfloat16
F32 = jnp.float32


def _dot(a, b):
    return lax.dot(a, b, preferred_element_type=F32)


def _dot_t(a, b):
    return lax.dot_general(a, b, (((1,), (1,)), ((), ())),
                           preferred_element_type=F32)


def _comm_body(x_ref, wdkv_ref, wuk_ref, wuv_ref, kv_ref, rs_recv,
               rs_send_sems, rs_recv_sems, ag_send_sems, ag_recv_sems):
    my = lax.axis_index("i")
    left = lax.rem(my + N_DEV - 1, N_DEV)
    right = lax.rem(my + 1, N_DEV)

    wdkv = wdkv_ref[:].astype(BF)
    wuk = wuk_ref[:].astype(BF)
    wuv = wuv_ref[:].astype(BF)
    for b in range(B):
        xb = x_ref[b].astype(BF)
        cb = _dot(xb, wdkv).astype(BF)
        kb = _dot(cb, wuk).astype(BF)
        vb = _dot(cb, wuv).astype(BF)
        n_chunks = S // CHUNK
        kv_ref[pl.ds(b * n_chunks, n_chunks)] = kb.reshape(n_chunks, CHUNK, D)
        kv_ref[pl.ds((2 + b) * n_chunks, n_chunks)] = vb.reshape(
            n_chunks, CHUNK, D)

    barrier_sem = pltpu.get_barrier_semaphore()
    for nbr in (left, right):
        pl.semaphore_signal(barrier_sem, inc=1, device_id=(nbr,),
                            device_id_type=pl.DeviceIdType.MESH)
    pl.semaphore_wait(barrier_sem, 2)

    for s in range(N_DEV - 1):
        send_idx = lax.rem(my - s + N_DEV, N_DEV)
        recv_idx = lax.rem(my - s - 1 + N_DEV, N_DEV)
        rdma = pltpu.make_async_remote_copy(
            src_ref=kv_ref.at[send_idx],
            dst_ref=rs_recv.at[s],
            send_sem=rs_send_sems.at[s],
            recv_sem=rs_recv_sems.at[s],
            device_id=(right,),
            device_id_type=pl.DeviceIdType.MESH,
        )
        rdma.start()
        rdma.wait()
        kv_ref[recv_idx] = (kv_ref[recv_idx].astype(F32)
                            + rs_recv[s].astype(F32)).astype(BF)

    for s in range(N_DEV - 1):
        send_idx = lax.rem(my + 1 - s + N_DEV, N_DEV)
        rdma = pltpu.make_async_remote_copy(
            src_ref=kv_ref.at[send_idx],
            dst_ref=kv_ref.at[send_idx],
            send_sem=ag_send_sems.at[s],
            recv_sem=ag_recv_sems.at[s],
            device_id=(right,),
            device_id_type=pl.DeviceIdType.MESH,
        )
        rdma.start()
        rdma.wait()


def _allreduce_kv(x, wdkv, wuk, wuv):
    return pl.pallas_call(
        _comm_body,
        out_shape=jax.ShapeDtypeStruct((N_DEV, CHUNK, D), BF),
        in_specs=[pl.BlockSpec(memory_space=pltpu.VMEM)] * 4,
        out_specs=pl.BlockSpec(memory_space=pltpu.VMEM),
        scratch_shapes=[
            pltpu.VMEM((N_DEV - 1, CHUNK, D), BF),
            pltpu.SemaphoreType.DMA((N_DEV - 1,)),
            pltpu.SemaphoreType.DMA((N_DEV - 1,)),
            pltpu.SemaphoreType.DMA((N_DEV - 1,)),
            pltpu.SemaphoreType.DMA((N_DEV - 1,)),
        ],
        compiler_params=pltpu.CompilerParams(
            collective_id=0, vmem_limit_bytes=60 * 1024 * 1024),
    )(x, wdkv, wuk, wuv)


def _attn_body(x_ref, wq_ref, wqr_ref, wkr_ref, kv_ref, o_ref,
               q_scr, qr_scr, kr_scr):
    wq = wq_ref[:].astype(BF)
    wqr = wqr_ref[:].astype(BF)
    wkr = wkr_ref[:].astype(BF)
    for b in range(B):
        xb = x_ref[b].astype(BF)
        q_scr[...] = _dot(xb, wq).astype(BF)
        qr_scr[...] = _dot(xb, wqr).astype(BF)
        kr_scr[...] = _dot(xb, wkr).astype(BF)
        kr = kr_scr[...]
        for h in range(H):
            qh = q_scr[:, h * Dh:(h + 1) * Dh]
            kh = kv_ref[b, :, h * Dh:(h + 1) * Dh]
            qrh = qr_scr[:, h * Dr:(h + 1) * Dr]
            sc = (_dot_t(qh, kh) + _dot_t(qrh, kr)) * SCALE
            m = jnp.max(sc, axis=-1, keepdims=True)
            p = jnp.exp(sc - m)
            p = p / jnp.sum(p, axis=-1, keepdims=True)
            vh = kv_ref[2 + b, :, h * Dh:(h + 1) * Dh]
            o_ref[b, :, h * Dh:(h + 1) * Dh] = _dot(p.astype(BF),
                                                    vh).astype(BF)


def _attention(x, wq, wqr, wkr, kv):
    return pl.pallas_call(
        _attn_body,
        out_shape=jax.ShapeDtypeStruct((B, S, H * Dh), BF),
        in_specs=[pl.BlockSpec(memory_space=pltpu.VMEM)] * 5,
        out_specs=pl.BlockSpec(memory_space=pltpu.VMEM),
        scratch_shapes=[
            pltpu.VMEM((S, H * Dh), BF),
            pltpu.VMEM((S, H * Dr), BF),
            pltpu.VMEM((S, Dr), BF),
        ],
        compiler_params=pltpu.CompilerParams(
            vmem_limit_bytes=60 * 1024 * 1024),
    )(x, wq, wqr, wkr, kv)


def _proj_body(o_ref, wo_ref, out_ref):
    wo = wo_ref[:].astype(BF)
    for b in range(B):
        out_ref[b] = _dot(o_ref[b], wo)


def _out_proj(o, wo):
    return pl.pallas_call(
        _proj_body,
        out_shape=jax.ShapeDtypeStruct((B, S, D), F32),
        in_specs=[pl.BlockSpec(memory_space=pltpu.VMEM)] * 2,
        out_specs=pl.BlockSpec(memory_space=pltpu.VMEM),
        compiler_params=pltpu.CompilerParams(
            vmem_limit_bytes=60 * 1024 * 1024),
    )(o, wo)


def kernel(x, Wdkv, Wuk, Wuv, Wq, Wqr, Wkr, Wo):
    kv = _allreduce_kv(x, Wdkv, Wuk, Wuv)
    kv = kv.reshape(4, S, D)
    o = _attention(x, Wq, Wqr, Wkr, kv)
    return _out_proj(o, Wo)


# device time: 218376 ns/iter; 1.7490x vs baseline; 1.7490x over previous
import jax
import jax.numpy as jnp
from jax import lax
from jax.experimental import pallas as pl
from jax.experimental.pallas import tpu as pltpu

N_DEV = 32
B, S, D = 2, 512, 2048
H, Dh, Dr = 16, 128, 32
NG = 8
GROUP = 256
HALF = GROUP // 2
SCALE = (Dh + Dr) ** -0.5

BF = jnp.bfloat16
F32 = jnp.float32

_PERM = [0, 1, 2, 5, 6, 7, 4, 3]
_INV = [0, 1, 2, 7, 6, 3, 4, 5]
_ZP1 = [1, 0, 3, 2]
_ZP2 = [2, 3, 0, 1]


def _lut(idx, table):
    out = jnp.int32(table[0])
    for i, v in enumerate(table[1:], 1):
        out = jnp.where(idx == i, jnp.int32(v), out)
    return out


def _dot(a, b):
    return lax.dot(a, b, preferred_element_type=F32)


def _dot_t(a, b):
    return lax.dot_general(a, b, (((1,), (1,)), ((), ())),
                           preferred_element_type=F32)


def _comm_body(x_ref, wdkv_ref, wuk_ref, wuv_ref, wq_ref, wqr_ref, wkr_ref,
               kv_ref, q_ref, qr_ref, kr_ref,
               p1_recv, z_recv,
               p1_ssem, p1_rsem, z_ssem, z_rsem, p3_ssem, p3_rsem):
    my = lax.axis_index("i")
    p = my // NG
    w = lax.rem(my, NG)
    r = _lut(w, _INV)
    right = p * NG + _lut(lax.rem(r + 1, NG), _PERM)
    left = p * NG + _lut(lax.rem(r + NG - 1, NG), _PERM)
    z1 = _lut(p, _ZP1) * NG + w
    z2 = _lut(p, _ZP2) * NG + w

    wdkv = wdkv_ref[:].astype(BF)
    wuk = wuk_ref[:].astype(BF)
    wuv = wuv_ref[:].astype(BF)
    for b in range(B):
        xb = x_ref[b].astype(BF)
        cb = _dot(xb, wdkv).astype(BF)
        kv_ref[pl.ds(2 * b, 2)] = _dot(cb, wuk).astype(BF).reshape(
            2, GROUP, D)
        kv_ref[pl.ds(4 + 2 * b, 2)] = _dot(cb, wuv).astype(BF).reshape(
            2, GROUP, D)

    pieces = []
    QS = (H * Dh) // 4

    def q_piece(b, j):
        def run():
            xb = x_ref[b].astype(BF)
            wq = wq_ref[:, pl.ds(j * QS, QS)].astype(BF)
            q_ref[b, :, pl.ds(j * QS, QS)] = _dot(xb, wq).astype(BF)
        return run

    def qr_piece(b):
        def run():
            xb = x_ref[b].astype(BF)
            qr_ref[b] = _dot(xb, wqr_ref[:].astype(BF)).astype(BF)
            kr_ref[b] = _dot(xb, wkr_ref[:].astype(BF)).astype(BF)
        return run

    for b in range(B):
        for j in range(4):
            pieces.append(q_piece(b, j))
        pieces.append(qr_piece(b))
    pieces.reverse()

    def overlap():
        if pieces:
            pieces.pop()()

    barrier_sem = pltpu.get_barrier_semaphore()
    for nbr in (left, right, z1, z2):
        pl.semaphore_signal(barrier_sem, inc=1, device_id=(nbr,),
                            device_id_type=pl.DeviceIdType.MESH)
    pl.semaphore_wait(barrier_sem, 4)

    def add_half(g, lo, buf):
        dst = kv_ref.at[g, pl.ds(lo, HALF)]
        dst[...] = (dst[...].astype(F32) + buf.astype(F32)).astype(BF)

    for s in range(NG - 1):
        g_cw = lax.rem(r - s + NG, NG)
        g_ccw = lax.rem(r + s, NG)
        cw = pltpu.make_async_remote_copy(
            src_ref=kv_ref.at[g_cw, pl.ds(0, HALF)],
            dst_ref=p1_recv.at[s, 0],
            send_sem=p1_ssem.at[s, 0], recv_sem=p1_rsem.at[s, 0],
            device_id=(right,), device_id_type=pl.DeviceIdType.MESH)
        ccw = pltpu.make_async_remote_copy(
            src_ref=kv_ref.at[g_ccw, pl.ds(HALF, HALF)],
            dst_ref=p1_recv.at[s, 1],
            send_sem=p1_ssem.at[s, 1], recv_sem=p1_rsem.at[s, 1],
            device_id=(left,), device_id_type=pl.DeviceIdType.MESH)
        cw.start()
        ccw.start()
        overlap()
        cw.wait()
        ccw.wait()
        add_half(lax.rem(r - s - 1 + NG, NG), 0, p1_recv[s, 0])
        add_half(lax.rem(r + s + 1, NG), HALF, p1_recv[s, 1])

    ga = lax.rem(r + 1, NG)
    gb = lax.rem(r + NG - 1, NG)

    for s, tgt in enumerate((z1, z2)):
        ra = pltpu.make_async_remote_copy(
            src_ref=kv_ref.at[ga, pl.ds(0, HALF)],
            dst_ref=z_recv.at[s, 0],
            send_sem=z_ssem.at[s, 0], recv_sem=z_rsem.at[s, 0],
            device_id=(tgt,), device_id_type=pl.DeviceIdType.MESH)
        rb = pltpu.make_async_remote_copy(
            src_ref=kv_ref.at[gb, pl.ds(HALF, HALF)],
            dst_ref=z_recv.at[s, 1],
            send_sem=z_ssem.at[s, 1], recv_sem=z_rsem.at[s, 1],
            device_id=(tgt,), device_id_type=pl.DeviceIdType.MESH)
        ra.start()
        rb.start()
        overlap()
        ra.wait()
        rb.wait()
        add_half(ga, 0, z_recv[s, 0])
        add_half(gb, HALF, z_recv[s, 1])

    for s in range(NG - 1):
        g_scw = lax.rem(r + 1 - s + NG, NG)
        g_rcw = lax.rem(r - s + NG, NG)
        g_sccw = lax.rem(r - 1 + s + NG, NG)
        g_rccw = lax.rem(r + s, NG)
        cw = pltpu.make_async_remote_copy(
            src_ref=kv_ref.at[g_scw, pl.ds(0, HALF)],
            dst_ref=kv_ref.at[g_scw, pl.ds(0, HALF)],
            send_sem=p3_ssem.at[s, 0], recv_sem=p3_rsem.at[s, 0],
            device_id=(right,), device_id_type=pl.DeviceIdType.MESH)
        ccw = pltpu.make_async_remote_copy(
            src_ref=kv_ref.at[g_sccw, pl.ds(HALF, HALF)],
            dst_ref=kv_ref.at[g_sccw, pl.ds(HALF, HALF)],
            send_sem=p3_ssem.at[s, 1], recv_sem=p3_rsem.at[s, 1],
            device_id=(left,), device_id_type=pl.DeviceIdType.MESH)
        cw.start()
        ccw.start()
        overlap()
        cw.wait()
        ccw.wait()

    while pieces:
        pieces.pop()()


def _comm_q(x, wdkv, wuk, wuv, wq, wqr, wkr):
    return pl.pallas_call(
        _comm_body,
        out_shape=(
            jax.ShapeDtypeStruct((NG, GROUP, D), BF),
            jax.ShapeDtypeStruct((B, S, H * Dh), BF),
            jax.ShapeDtypeStruct((B, S, H * Dr), BF),
            jax.ShapeDtypeStruct((B, S, Dr), BF),
        ),
        in_specs=[pl.BlockSpec(memory_space=pltpu.VMEM)] * 7,
        out_specs=(pl.BlockSpec(memory_space=pltpu.VMEM),) * 4,
        scratch_shapes=[
            pltpu.VMEM((NG - 1, 2, HALF, D), BF),
            pltpu.VMEM((2, 2, HALF, D), BF),
            pltpu.SemaphoreType.DMA((NG - 1, 2)),
            pltpu.SemaphoreType.DMA((NG - 1, 2)),
            pltpu.SemaphoreType.DMA((2, 2)),
            pltpu.SemaphoreType.DMA((2, 2)),
            pltpu.SemaphoreType.DMA((NG - 1, 2)),
            pltpu.SemaphoreType.DMA((NG - 1, 2)),
        ],
        compiler_params=pltpu.CompilerParams(
            collective_id=0, vmem_limit_bytes=62 * 1024 * 1024),
    )(x, wdkv, wuk, wuv, wq, wqr, wkr)


def _attn_body(q_ref, qr_ref, kr_ref, kv_ref, wo_ref, out_ref, o_scr):
    for b in range(B):
        k = kv_ref[pl.ds(2 * b, 2)].reshape(S, D)
        v = kv_ref[pl.ds(4 + 2 * b, 2)].reshape(S, D)
        kr = kr_ref[b]
        for h in range(H):
            qh = q_ref[b, :, h * Dh:(h + 1) * Dh]
            qrh = qr_ref[b, :, h * Dr:(h + 1) * Dr]
            sc = (_dot_t(qh, k[:, h * Dh:(h + 1) * Dh])
                  + _dot_t(qrh, kr)) * SCALE
            m = jnp.max(sc, axis=-1, keepdims=True)
            pr = jnp.exp(sc - m)
            pr = pr / jnp.sum(pr, axis=-1, keepdims=True)
            o_scr[:, h * Dh:(h + 1) * Dh] = _dot(
                pr.astype(BF), v[:, h * Dh:(h + 1) * Dh]).astype(BF)
        acc = jnp.zeros((S, D), F32)
        for j in range(4):
            wo = wo_ref[pl.ds(j * 512, 512), :].astype(BF)
            acc = acc + _dot(o_scr[:, j * 512:(j + 1) * 512], wo)
        out_ref[b] = acc


def _attention(q, qr, kr, kv, wo):
    return pl.pallas_call(
        _attn_body,
        out_shape=jax.ShapeDtypeStruct((B, S, D), F32),
        in_specs=[pl.BlockSpec(memory_space=pltpu.VMEM)] * 5,
        out_specs=pl.BlockSpec(memory_space=pltpu.VMEM),
        scratch_shapes=[pltpu.VMEM((S, H * Dh), BF)],
        compiler_params=pltpu.CompilerParams(
            vmem_limit_bytes=62 * 1024 * 1024),
    )(q, qr, kr, kv, wo)


def kernel(x, Wdkv, Wuk, Wuv, Wq, Wqr, Wkr, Wo):
    kv, q, qr, kr = _comm_q(x, Wdkv, Wuk, Wuv, Wq, Wqr, Wkr)
    return _attention(q, qr, kr, kv, Wo)
